# Initial kernel scaffold; baseline (speedup 1.0000x reference)
import jax
import jax.numpy as jnp
from jax.experimental import pallas as pl


def kernel(x, edge_index, W1, a_src1, a_dst1, b1, g1, be1, W2, a_src2, a_dst2, b2, g2, be2, W3, a_src3, a_dst3, b3, g3, be3):
    raise NotImplementedError("write your pallas kernel here")


# speedup vs baseline: 25.3633x; 25.3633x over previous
import functools

import jax
import jax.numpy as jnp
from jax import lax
from jax.experimental import pallas as pl
from jax.experimental.pallas import tpu as pltpu
from jax.experimental.pallas import tpu_sc as plsc

N = 10000
NP = 10240
E = 640000
NEG = 0.2
BN_INV = 1.0 / (1.0 + 1e-5) ** 0.5
BN_B = 640
NB = NP // BN_B
NT = 16
RPT = NP // NT
EPT4 = E // NT
EPT1 = E // (2 * NT)
EB = 80
F32 = jnp.float32

_HIGH = jax.lax.Precision.HIGHEST


def _dot(a, b):
    return jax.lax.dot_general(a, b, (((1,), (0,)), ((), ())),
                               preferred_element_type=F32, precision=_HIGH)


def _elu(v):
    return jnp.where(v > 0, v, jnp.exp(v) - 1.0)


def _tc1_body(x_ref, w_ref, asrc_ref, adst_ref, h_ref, als_ref, ald_ref):
    x = x_ref[...]
    for h in range(4):
        hh = _dot(x, w_ref[h])
        h_ref[h] = hh
        als_ref[h] = jnp.sum(hh * asrc_ref[h], axis=1)
        ald_ref[h] = jnp.sum(hh * adst_ref[h], axis=1)


def _tc1(x, w, asrc, adst):
    return pl.pallas_call(
        _tc1_body,
        grid=(NB,),
        in_specs=[
            pl.BlockSpec((BN_B, 128), lambda i: (i, 0)),
            pl.BlockSpec((4, 128, 128), lambda i: (0, 0, 0)),
            pl.BlockSpec((4, 128), lambda i: (0, 0)),
            pl.BlockSpec((4, 128), lambda i: (0, 0)),
        ],
        out_specs=[
            pl.BlockSpec((4, BN_B, 128), lambda i: (0, i, 0)),
            pl.BlockSpec((4, BN_B), lambda i: (0, i)),
            pl.BlockSpec((4, BN_B), lambda i: (0, i)),
        ],
        out_shape=[
            jax.ShapeDtypeStruct((4, NP, 128), F32),
            jax.ShapeDtypeStruct((4, NP), F32),
            jax.ShapeDtypeStruct((4, NP), F32),
        ],
    )(x, w, asrc, adst)


def _tc2_body(agg_ref, den_ref, b_ref, g_ref, be_ref, w_ref, asrc_ref,
              adst_ref, h2_ref, als_ref, ald_ref):
    x2 = []
    for h in range(4):
        d = jnp.sum(den_ref[h], axis=0)
        v = agg_ref[h] / (d[:, None] + 1e-16) + b_ref[h]
        v = v * (g_ref[h] * BN_INV) + be_ref[h]
        x2.append(_elu(v))
    for ho in range(4):
        acc = _dot(x2[0], w_ref[0, ho])
        for hi in range(1, 4):
            acc = acc + _dot(x2[hi], w_ref[hi, ho])
        h2_ref[ho] = acc
        als_ref[ho] = jnp.sum(acc * asrc_ref[ho], axis=1)
        ald_ref[ho] = jnp.sum(acc * adst_ref[ho], axis=1)


def _tc2(agg, den, b, g, be, w, asrc, adst):
    return pl.pallas_call(
        _tc2_body,
        grid=(NB,),
        in_specs=[
            pl.BlockSpec((4, BN_B, 128), lambda i: (0, i, 0)),
            pl.BlockSpec((4, NT, BN_B), lambda i: (0, 0, i)),
            pl.BlockSpec((4, 128), lambda i: (0, 0)),
            pl.BlockSpec((4, 128), lambda i: (0, 0)),
            pl.BlockSpec((4, 128), lambda i: (0, 0)),
            pl.BlockSpec((4, 4, 128, 128), lambda i: (0, 0, 0, 0)),
            pl.BlockSpec((4, 128), lambda i: (0, 0)),
            pl.BlockSpec((4, 128), lambda i: (0, 0)),
        ],
        out_specs=[
            pl.BlockSpec((4, BN_B, 128), lambda i: (0, i, 0)),
            pl.BlockSpec((4, BN_B), lambda i: (0, i)),
            pl.BlockSpec((4, BN_B), lambda i: (0, i)),
        ],
        out_shape=[
            jax.ShapeDtypeStruct((4, NP, 128), F32),
            jax.ShapeDtypeStruct((4, NP), F32),
            jax.ShapeDtypeStruct((4, NP), F32),
        ],
    )(agg, den, b, g, be, w, asrc, adst)


def _tc3_body(agg_ref, den_ref, b_ref, g_ref, be_ref, w_ref, asrc_ref,
              adst_ref, h3_ref, als_ref, ald_ref):
    acc = None
    for h in range(4):
        d = jnp.sum(den_ref[h], axis=0)
        v = agg_ref[h] / (d[:, None] + 1e-16) + b_ref[h]
        v = v * (g_ref[h] * BN_INV) + be_ref[h]
        v = _elu(v)
        p = _dot(v, w_ref[h])
        acc = p if acc is None else acc + p
    h3_ref[...] = acc
    als_ref[0] = jnp.sum(acc * asrc_ref[...], axis=1)
    ald_ref[0] = jnp.sum(acc * adst_ref[...], axis=1)


def _tc3(agg, den, b, g, be, w, asrc, adst):
    return pl.pallas_call(
        _tc3_body,
        grid=(NB,),
        in_specs=[
            pl.BlockSpec((4, BN_B, 128), lambda i: (0, i, 0)),
            pl.BlockSpec((4, NT, BN_B), lambda i: (0, 0, i)),
            pl.BlockSpec((4, 128), lambda i: (0, 0)),
            pl.BlockSpec((4, 128), lambda i: (0, 0)),
            pl.BlockSpec((4, 128), lambda i: (0, 0)),
            pl.BlockSpec((4, 128, 64), lambda i: (0, 0, 0)),
            pl.BlockSpec((1, 64), lambda i: (0, 0)),
            pl.BlockSpec((1, 64), lambda i: (0, 0)),
        ],
        out_specs=[
            pl.BlockSpec((BN_B, 64), lambda i: (i, 0)),
            pl.BlockSpec((1, BN_B), lambda i: (0, i)),
            pl.BlockSpec((1, BN_B), lambda i: (0, i)),
        ],
        out_shape=[
            jax.ShapeDtypeStruct((NP, 64), F32),
            jax.ShapeDtypeStruct((1, NP), F32),
            jax.ShapeDtypeStruct((1, NP), F32),
        ],
    )(agg, den, b, g, be, w, asrc, adst)


def _tc4_body(agg_ref, den_ref, b_ref, g_ref, be_ref, out_ref):
    d = jnp.sum(den_ref[0], axis=0) + jnp.sum(den_ref[1], axis=0)
    v = (agg_ref[0] + agg_ref[1]) / (d[:, None] + 1e-16) + b_ref[...]
    out_ref[...] = v * (g_ref[...] * BN_INV) + be_ref[...]


def _tc4(agg, den, b, g, be):
    return pl.pallas_call(
        _tc4_body,
        grid=(NB,),
        in_specs=[
            pl.BlockSpec((2, BN_B, 64), lambda i: (0, i, 0)),
            pl.BlockSpec((2, NT, BN_B), lambda i: (0, 0, i)),
            pl.BlockSpec((1, 64), lambda i: (0, 0)),
            pl.BlockSpec((1, 64), lambda i: (0, 0)),
            pl.BlockSpec((1, 64), lambda i: (0, 0)),
        ],
        out_specs=pl.BlockSpec((BN_B, 64), lambda i: (i, 0)),
        out_shape=jax.ShapeDtypeStruct((NP, 64), F32),
    )(agg, den, b, g, be)


_MESH = plsc.VectorSubcoreMesh(core_axis_name="c", subcore_axis_name="s",
                               num_cores=2, num_subcores=NT)


def _edge_round(head, table, als_hbm, ald_hbm, src_hbm, dst_hbm, agg_row,
                den_row, zrows_hbm, zden_hbm, acc_sh, als_v, ald_v, den_v,
                src_v, dst_v, idx_v, w_v, rows_v, sem, s, ebase, nbatch,
                nvec, tab_off):
    pltpu.sync_copy(zrows_hbm, acc_sh.at[pl.ds(s * RPT, RPT)])
    pltpu.sync_copy(als_hbm.at[head], als_v)
    pltpu.sync_copy(ald_hbm.at[head], ald_v)
    pltpu.sync_copy(zden_hbm, den_v)
    plsc.subcore_barrier()

    def batch(b, _):
        base = ebase + b * EB
        pltpu.sync_copy(src_hbm.at[pl.ds(base, EB)], src_v)
        pltpu.sync_copy(dst_hbm.at[pl.ds(base, EB)], dst_v)

        def wstep(j, _):
            sl = pl.ds(j * 16, 16)
            si = src_v[sl]
            di = dst_v[sl]
            e = plsc.load_gather(als_v, [si]) + plsc.load_gather(ald_v, [di])
            e = jnp.where(e > 0, e, NEG * e)
            w = jnp.exp(e)
            w_v[sl] = w
            plsc.addupdate_scatter(den_v, [di], w)
            idx_v[sl] = si + tab_off
            return 0

        lax.fori_loop(0, EB // 16, wstep, 0)
        pltpu.async_copy(table.at[idx_v], rows_v, sem).wait()

        def scale(j, _):
            wj = plsc.load_gather(w_v, [jnp.zeros((16,), jnp.int32) + j])
            for k in range(nvec):
                sk = pl.ds(k * 16, 16)
                rows_v[j, sk] = rows_v[j, sk] * wj
            return 0

        lax.fori_loop(0, EB, scale, 0)
        pltpu.sync_copy(rows_v, acc_sh.at[dst_v], add=True)
        return 0

    lax.fori_loop(0, nbatch, batch, 0)
    plsc.subcore_barrier()
    pltpu.sync_copy(acc_sh.at[pl.ds(s * RPT, RPT)],
                    agg_row.at[pl.ds(s * RPT, RPT)])
    pltpu.sync_copy(den_v, den_row.at[s])
    plsc.subcore_barrier()


def _sc4_body(table, als_hbm, ald_hbm, src_hbm, dst_hbm, zrows_hbm, zden_hbm,
              agg_out, den_out, acc_sh, als_v, ald_v, den_v, src_v, dst_v,
              idx_v, w_v, rows_v, sem):
    c = lax.axis_index("c")
    s = lax.axis_index("s")
    for r in range(2):
        head = c * 2 + r
        _edge_round(head, table, als_hbm, ald_hbm, src_hbm, dst_hbm,
                    agg_out.at[head], den_out.at[head], zrows_hbm, zden_hbm,
                    acc_sh, als_v, ald_v, den_v, src_v, dst_v, idx_v, w_v,
                    rows_v, sem, s, s * EPT4, EPT4 // EB, 8, head * NP)


def _sc4(table, als, ald, src, dst, zrows, zden):
    return pl.kernel(
        _sc4_body,
        out_type=(jax.ShapeDtypeStruct((4, NP, 128), F32),
                  jax.ShapeDtypeStruct((4, NT, NP), F32)),
        mesh=_MESH,
        compiler_params=pltpu.CompilerParams(needs_layout_passes=False, use_tc_tiling_on_sc=False),
        scratch_types=[
            pltpu.VMEM_SHARED((NP, 128), F32),
            pltpu.VMEM((NP,), F32),
            pltpu.VMEM((NP,), F32),
            pltpu.VMEM((NP,), F32),
            pltpu.VMEM((EB,), jnp.int32),
            pltpu.VMEM((EB,), jnp.int32),
            pltpu.VMEM((EB,), jnp.int32),
            pltpu.VMEM((EB,), F32),
            pltpu.VMEM((EB, 128), F32),
            pltpu.SemaphoreType.DMA,
        ],
    )(table, als, ald, src, dst, zrows, zden)


def _sc1_body(table, als_hbm, ald_hbm, src_hbm, dst_hbm, zrows_hbm, zden_hbm,
              agg_out, den_out, acc_sh, als_v, ald_v, den_v, src_v, dst_v,
              idx_v, w_v, rows_v, sem):
    c = lax.axis_index("c")
    s = lax.axis_index("s")
    _edge_round(0, table, als_hbm, ald_hbm, src_hbm, dst_hbm, agg_out.at[c],
                den_out.at[c], zrows_hbm, zden_hbm, acc_sh, als_v, ald_v,
                den_v, src_v, dst_v, idx_v, w_v, rows_v, sem, s,
                (c * NT + s) * EPT1, EPT1 // EB, 4, 0)


def _sc1(table, als, ald, src, dst, zrows, zden):
    return pl.kernel(
        _sc1_body,
        out_type=(jax.ShapeDtypeStruct((2, NP, 64), F32),
                  jax.ShapeDtypeStruct((2, NT, NP), F32)),
        mesh=_MESH,
        compiler_params=pltpu.CompilerParams(needs_layout_passes=False, use_tc_tiling_on_sc=False),
        scratch_types=[
            pltpu.VMEM_SHARED((NP, 64), F32),
            pltpu.VMEM((NP,), F32),
            pltpu.VMEM((NP,), F32),
            pltpu.VMEM((NP,), F32),
            pltpu.VMEM((EB,), jnp.int32),
            pltpu.VMEM((EB,), jnp.int32),
            pltpu.VMEM((EB,), jnp.int32),
            pltpu.VMEM((EB,), F32),
            pltpu.VMEM((EB, 64), F32),
            pltpu.SemaphoreType.DMA,
        ],
    )(table, als, ald, src, dst, zrows, zden)


def kernel(x, edge_index, W1, a_src1, a_dst1, b1, g1, be1, W2, a_src2,
           a_dst2, b2, g2, be2, W3, a_src3, a_dst3, b3, g3, be3):
    src = edge_index[0]
    dst = edge_index[1]
    xp = jnp.pad(x, ((0, NP - N), (0, 0)))

    w1r = jnp.transpose(W1.reshape(128, 4, 128), (1, 0, 2))
    w2r = jnp.transpose(W2.reshape(4, 128, 4, 128), (0, 2, 1, 3))
    w3r = W3.reshape(4, 128, 64)
    as1, ad1 = a_src1.reshape(4, 128), a_dst1.reshape(4, 128)
    as2, ad2 = a_src2.reshape(4, 128), a_dst2.reshape(4, 128)
    as3, ad3 = a_src3.reshape(1, 64), a_dst3.reshape(1, 64)
    b1r, g1r, be1r = b1.reshape(4, 128), g1.reshape(4, 128), be1.reshape(4, 128)
    b2r, g2r, be2r = b2.reshape(4, 128), g2.reshape(4, 128), be2.reshape(4, 128)
    b3r, g3r, be3r = b3.reshape(1, 64), g3.reshape(1, 64), be3.reshape(1, 64)

    zden = jnp.zeros((NP,), F32)
    zrows = jnp.zeros((RPT, 128), F32)
    zrows3 = jnp.zeros((RPT, 64), F32)

    h1, als1, ald1 = _tc1(xp, w1r, as1, ad1)
    agg1, den1 = _sc4(h1.reshape(4 * NP, 128), als1, ald1, src, dst,
                      zrows, zden)
    h2, als2, ald2 = _tc2(agg1, den1, b1r, g1r, be1r, w2r, as2, ad2)
    agg2, den2 = _sc4(h2.reshape(4 * NP, 128), als2, ald2, src, dst,
                      zrows, zden)
    h3, als3, ald3 = _tc3(agg2, den2, b2r, g2r, be2r, w3r, as3, ad3)
    agg3, den3 = _sc1(h3, als3, ald3, src, dst, zrows3, zden)
    outp = _tc4(agg3, den3, b3r, g3r, be3r)
    return outp[:N]
